# baseline (speedup 1.0000x reference)
import jax
import jax.numpy as jnp
from jax import lax
from jax.experimental import pallas as pl
from jax.experimental.pallas import tpu as pltpu
from jax.experimental.pallas import tpu_sc as plsc

_VOCAB = 100000
_D = 64
_NB = 4096
_NT = 200
_NC = 2
_NS = 16
_NW = _NC * _NS
_BL = 128
_LANES = 16


def _body(x_hbm, w_hbm, out_hbm, xstg, idx_v, rows0, rows1, tr0, tr1,
          gsem0, gsem1, wsem0, wsem1):
    wid = lax.axis_index("s") * _NC + lax.axis_index("c")
    b0 = wid * _BL
    iota = lax.iota(jnp.int32, _LANES)

    pltpu.sync_copy(x_hbm.at[pl.ds(b0, _BL)], xstg.at[:, pl.ds(0, _NT)])

    @plsc.parallel_loop(0, _NT * (_BL // _LANES), unroll=8)
    def _cols(j):
        t = j >> 3
        k = j & 7
        tcol = jnp.full((_LANES,), t, jnp.int32)
        v = plsc.load_gather(xstg, [iota + k * _LANES, tcol])
        pos = (t << 7) | (k << 4)
        idx_v[pos >> 8, pl.ds(pos & 255, _LANES)] = v

    def fire(p, rows, gsem):
        pltpu.async_copy(w_hbm.at[idx_v.at[p]], rows, gsem)

    def drain(p, rows, gsem):
        pltpu.make_async_copy(w_hbm.at[idx_v.at[p]], rows, gsem).wait()

    def transpose(rows, h, tr):
        @plsc.parallel_loop(0, _BL // _LANES, unroll=4)
        def _tp(k):
            b_idx = iota + ((k << 4) + (h << 7))
            s_idx = iota + (k << 4)
            for d0 in range(_D):
                dvec = (iota + d0) & (_D - 1)
                v = plsc.load_gather(rows, [b_idx, dvec])
                plsc.store_scatter(tr, [dvec, s_idx], v)

    def put(t, tr, wsem):
        for dh in range(_D // 8):
            pltpu.async_copy(tr.at[pl.ds(8 * dh, 8)], out_hbm.at[t, dh, wid],
                             wsem)

    def wait_put(t, tr, wsem):
        for dh in range(_D // 8):
            pltpu.make_async_copy(tr.at[pl.ds(8 * dh, 8)],
                                  out_hbm.at[t, dh, wid], wsem).wait()

    fire(0, rows0, gsem0)
    fire(1, rows1, gsem1)

    _NP = _NT // 2

    @pl.loop(0, _NP // 2)
    def _block(u):
        p0 = 2 * u
        for j, (rows, gsem) in enumerate(((rows0, gsem0), (rows1, gsem1))):
            p = p0 + j
            drain(p, rows, gsem)
            for h in range(2):
                t = 2 * p + h
                trs, wsem = (tr0, wsem0) if h == 0 else (tr1, wsem1)

                @pl.when(t >= 2)
                def _():
                    wait_put(t - 2, trs, wsem)

                transpose(rows, h, trs)
                put(t, trs, wsem)

            @pl.when(u < _NP // 2 - 1)
            def _():
                fire(p + 2, rows, gsem)

    wait_put(_NT - 2, tr0, wsem0)
    wait_put(_NT - 1, tr1, wsem1)


@jax.jit
def _lookup(x, weight):
    mesh = plsc.VectorSubcoreMesh(
        core_axis_name="c", subcore_axis_name="s",
        num_cores=_NC, num_subcores=_NS,
    )
    return pl.kernel(
        _body,
        out_type=jax.ShapeDtypeStruct((_NT, _D // 8, _NB // _BL, 8, _BL),
                                      jnp.float32),
        mesh=mesh,
        scratch_types=[
            pltpu.VMEM((_BL, _NT + 1), jnp.int32),
            pltpu.VMEM((_NT // 2, 2 * _BL), jnp.int32),
            pltpu.VMEM((2 * _BL, _D), jnp.float32),
            pltpu.VMEM((2 * _BL, _D), jnp.float32),
            pltpu.VMEM((_D, _BL), jnp.float32),
            pltpu.VMEM((_D, _BL), jnp.float32),
            pltpu.SemaphoreType.DMA,
            pltpu.SemaphoreType.DMA,
            pltpu.SemaphoreType.DMA,
            pltpu.SemaphoreType.DMA,
        ],
        compiler_params=pltpu.CompilerParams(
            use_tc_tiling_on_sc=False, needs_layout_passes=False),
    )(x, weight)


def kernel(x, weight):
    out5d = _lookup(x, weight)
    return jnp.transpose(out5d, (2, 4, 0, 1, 3)).reshape(_NB, _NT, _D)


# speedup vs baseline: 1.1477x; 1.1477x over previous
import jax
import jax.numpy as jnp
from jax import lax
from jax.experimental import pallas as pl
from jax.experimental.pallas import tpu as pltpu
from jax.experimental.pallas import tpu_sc as plsc

_VOCAB = 100000
_D = 64
_NB = 4096
_NT = 200
_NC = 2
_NS = 16
_NW = _NC * _NS
_BL = 128
_LANES = 16


def _body(x_hbm, w_hbm, out_hbm, xstg, idx_v, rows0, rows1, tr0, tr1,
          gsem0, gsem1, wsem0, wsem1):
    wid = lax.axis_index("s") * _NC + lax.axis_index("c")
    b0 = wid * _BL
    iota = lax.iota(jnp.int32, _LANES)

    pltpu.sync_copy(x_hbm.at[pl.ds(b0, _BL)], xstg.at[:, pl.ds(0, _NT)])

    @plsc.parallel_loop(0, _NT * (_BL // _LANES), unroll=8)
    def _cols(j):
        t = j >> 3
        k = j & 7
        tcol = jnp.full((_LANES,), t, jnp.int32)
        v = plsc.load_gather(xstg, [iota + k * _LANES, tcol])
        idx_v[t, pl.ds(k * _LANES, _LANES)] = v

    def fire(t, rows, gsem):
        pltpu.async_copy(w_hbm.at[idx_v.at[t]], rows, gsem)

    def drain(t, rows, gsem):
        pltpu.make_async_copy(w_hbm.at[idx_v.at[t]], rows, gsem).wait()

    def transpose(rows, tr):
        @plsc.parallel_loop(0, _BL // _LANES, unroll=4)
        def _tp(k):
            b_idx = iota + (k << 4)
            for d0 in range(_D):
                dvec = (iota + d0) & (_D - 1)
                v = plsc.load_gather(rows, [b_idx, dvec])
                plsc.store_scatter(tr, [dvec, b_idx], v)

    def put(t, tr, wsem):
        for dh in range(_D // 8):
            pltpu.async_copy(tr.at[pl.ds(8 * dh, 8)], out_hbm.at[t, dh, wid],
                             wsem)

    def wait_put(t, tr, wsem):
        for dh in range(_D // 8):
            pltpu.make_async_copy(tr.at[pl.ds(8 * dh, 8)],
                                  out_hbm.at[t, dh, wid], wsem).wait()

    fire(0, rows0, gsem0)
    fire(1, rows1, gsem1)

    @pl.loop(0, _NT // 2)
    def _block(u):
        t0 = 2 * u
        drain(t0, rows0, gsem0)

        @pl.when(u > 0)
        def _():
            wait_put(t0 - 2, tr0, wsem0)

        transpose(rows0, tr0)
        put(t0, tr0, wsem0)

        @pl.when(u < _NT // 2 - 1)
        def _():
            fire(t0 + 2, rows0, gsem0)

        drain(t0 + 1, rows1, gsem1)

        @pl.when(u > 0)
        def _():
            wait_put(t0 - 1, tr1, wsem1)

        transpose(rows1, tr1)
        put(t0 + 1, tr1, wsem1)

        @pl.when(u < _NT // 2 - 1)
        def _():
            fire(t0 + 3, rows1, gsem1)

    wait_put(_NT - 2, tr0, wsem0)
    wait_put(_NT - 1, tr1, wsem1)


@jax.jit
def _lookup(x, weight):
    mesh = plsc.VectorSubcoreMesh(
        core_axis_name="c", subcore_axis_name="s",
        num_cores=_NC, num_subcores=_NS,
    )
    return pl.kernel(
        _body,
        out_type=jax.ShapeDtypeStruct((_NT, _D // 8, _NB // _BL, 8, _BL),
                                      jnp.float32),
        mesh=mesh,
        scratch_types=[
            pltpu.VMEM((_BL, _NT + 1), jnp.int32),
            pltpu.VMEM((_NT, _BL), jnp.int32),
            pltpu.VMEM((_BL, _D), jnp.float32),
            pltpu.VMEM((_BL, _D), jnp.float32),
            pltpu.VMEM((_D, _BL), jnp.float32),
            pltpu.VMEM((_D, _BL), jnp.float32),
            pltpu.SemaphoreType.DMA,
            pltpu.SemaphoreType.DMA,
            pltpu.SemaphoreType.DMA,
            pltpu.SemaphoreType.DMA,
        ],
        compiler_params=pltpu.CompilerParams(
            use_tc_tiling_on_sc=False, needs_layout_passes=False),
    )(x, weight)


def kernel(x, weight):
    out5d = _lookup(x, weight)
    return jnp.transpose(out5d, (2, 4, 0, 1, 3)).reshape(_NB, _NT, _D)
